# baseline (speedup 1.0000x reference)
import functools

import jax
import jax.numpy as jnp
from jax import lax
from jax.experimental import pallas as pl
from jax.experimental.pallas import tpu as pltpu
from jax.experimental.pallas import tpu_sc as plsc

N_NODES = 100000
HIDDEN = 128
NUM_GRAPHS = 512
NW = 32
CHUNK = 3136
TAIL = N_NODES - (NW - 1) * CHUNK
CGRP = HIDDEN // 16
CNT_BASE = NUM_GRAPHS
DUMP_SEG = 576
SEGT = 592


def _sc_partials():
    mesh = plsc.VectorSubcoreMesh(core_axis_name="c", subcore_axis_name="s")

    @functools.partial(
        pl.kernel,
        mesh=mesh,
        out_type=jax.ShapeDtypeStruct((NW, SEGT, HIDDEN), jnp.float32),
        scratch_types=[
            pltpu.VMEM((CHUNK,), jnp.float32),
            pltpu.VMEM((CHUNK,), jnp.int32),
            pltpu.VMEM((SEGT, HIDDEN), jnp.float32),
            pltpu.VMEM((HIDDEN,), jnp.float32),
            pltpu.VMEM((HIDDEN,), jnp.float32),
        ],
    )
    def k(x_hbm, seg_hbm, w_hbm, b_hbm, psum_hbm,
          x_v, seg_v, acc_v, w_v, b_v):
        wid = lax.axis_index("s") * 2 + lax.axis_index("c")
        base = wid * CHUNK
        is_tail = wid == NW - 1
        zero16 = jnp.zeros((16,), jnp.float32)

        @pl.when(jnp.logical_not(is_tail))
        def _():
            pltpu.sync_copy(x_hbm.at[pl.ds(base, CHUNK)], x_v)
            pltpu.sync_copy(seg_hbm.at[pl.ds(base, CHUNK)], seg_v)

        @pl.when(is_tail)
        def _():
            pltpu.sync_copy(x_hbm.at[pl.ds(base, TAIL)], x_v.at[pl.ds(0, TAIL)])
            pltpu.sync_copy(seg_hbm.at[pl.ds(base, TAIL)],
                            seg_v.at[pl.ds(0, TAIL)])
            dump16 = jnp.full((16,), DUMP_SEG, jnp.int32)
            for t in range(TAIL, CHUNK, 16):
                x_v[pl.ds(t, 16)] = zero16
                seg_v[pl.ds(t, 16)] = dump16

        pltpu.sync_copy(w_hbm, w_v)
        pltpu.sync_copy(b_hbm, b_v)

        def zbody(r, carry):
            for j in range(CGRP):
                acc_v[r, pl.ds(16 * j, 16)] = zero16
            return carry

        lax.fori_loop(0, SEGT, zbody, 0)

        w_regs = [w_v[pl.ds(16 * j, 16)] for j in range(CGRP)]
        b_regs = [b_v[pl.ds(16 * j, 16)] for j in range(CGRP)]
        csix = jnp.full((16,), 0.0625, jnp.float32)

        ones16 = jnp.ones((16,), jnp.float32)

        def gbody(g, carry):
            x16 = x_v[pl.ds(g * 16, 16)]
            s16 = seg_v[pl.ds(g * 16, 16)]
            s_first = s16[0]
            s_last = s16[15]

            @pl.when(s_first == s_last)
            def _():
                for j in range(CGRP):
                    hs = [jnp.maximum(x16[lane] * w_regs[j] + b_regs[j], 0.0)
                          for lane in range(16)]
                    while len(hs) > 1:
                        hs = [hs[i] + hs[i + 1] for i in range(0, len(hs), 2)]
                    plsc.addupdate(acc_v.at[s_first, pl.ds(16 * j, 16)],
                                   hs[0])
                plsc.addupdate(
                    acc_v.at[CNT_BASE + (s_first >> 3),
                             pl.ds((s_first & 7) * 16, 16)],
                    ones16)

            @pl.when(s_first != s_last)
            def _():
                for lane in range(16):
                    xi = x16[lane]
                    si = s16[lane]
                    for j in range(CGRP):
                        h = jnp.maximum(xi * w_regs[j] + b_regs[j], 0.0)
                        plsc.addupdate(acc_v.at[si, pl.ds(16 * j, 16)], h)
                    plsc.addupdate(
                        acc_v.at[CNT_BASE + (si >> 3),
                                 pl.ds((si & 7) * 16, 16)],
                        csix)
            return carry

        lax.fori_loop(0, CHUNK // 16, gbody, 0)

        pltpu.sync_copy(acc_v, psum_hbm.at[wid])

    return k


def _combine(psum):
    def body(ps_ref, out_ref):
        s = jnp.sum(ps_ref[...], axis=0)
        sums = s[:NUM_GRAPHS]
        craw = s[CNT_BASE:CNT_BASE + NUM_GRAPHS // 8]
        rep = jnp.broadcast_to(
            craw[:, None, :], (NUM_GRAPHS // 8, 8, HIDDEN)
        ).reshape(NUM_GRAPHS, HIDDEN)
        lane_grp = lax.broadcasted_iota(jnp.int32, (NUM_GRAPHS, HIDDEN), 1) // 16
        seg_grp = lax.broadcasted_iota(jnp.int32, (NUM_GRAPHS, HIDDEN), 0) % 8
        cnt = jnp.sum(
            jnp.where(lane_grp == seg_grp, rep, 0.0), axis=1, keepdims=True)
        out_ref[...] = sums / jnp.maximum(cnt, 1.0)

    return pl.pallas_call(
        body,
        out_shape=jax.ShapeDtypeStruct((NUM_GRAPHS, HIDDEN), jnp.float32),
    )(psum)


def kernel(x, edge_index, batch, W, b):
    del edge_index
    x_flat = x.reshape(N_NODES)
    seg = batch.astype(jnp.int32)
    w_flat = W.reshape(HIDDEN)
    b_flat = b.reshape(HIDDEN)
    psum = _sc_partials()(x_flat, seg, w_flat, b_flat)
    return _combine(psum)


# speedup vs baseline: 1.1223x; 1.1223x over previous
import functools

import jax
import jax.numpy as jnp
from jax import lax
from jax.experimental import pallas as pl
from jax.experimental.pallas import tpu as pltpu
from jax.experimental.pallas import tpu_sc as plsc

N_NODES = 100000
HIDDEN = 128
NUM_GRAPHS = 512
NW = 32
CHUNK = 3136
TAIL = N_NODES - (NW - 1) * CHUNK
CGRP = HIDDEN // 16
CNT_BASE = NUM_GRAPHS
DUMP_SEG = 576
SEGT = 592


def _sc_partials():
    mesh = plsc.VectorSubcoreMesh(core_axis_name="c", subcore_axis_name="s")

    @functools.partial(
        pl.kernel,
        mesh=mesh,
        out_type=jax.ShapeDtypeStruct((NW, SEGT, HIDDEN), jnp.float32),
        scratch_types=[
            pltpu.VMEM((CHUNK,), jnp.float32),
            pltpu.VMEM((CHUNK,), jnp.int32),
            pltpu.VMEM((SEGT, HIDDEN), jnp.float32),
            pltpu.VMEM((HIDDEN,), jnp.float32),
            pltpu.VMEM((HIDDEN,), jnp.float32),
        ],
    )
    def k(x_hbm, seg_hbm, w_hbm, b_hbm, psum_hbm,
          x_v, seg_v, acc_v, w_v, b_v):
        wid = lax.axis_index("s") * 2 + lax.axis_index("c")
        base = wid * CHUNK
        is_tail = wid == NW - 1
        zero16 = jnp.zeros((16,), jnp.float32)

        @pl.when(jnp.logical_not(is_tail))
        def _():
            pltpu.sync_copy(x_hbm.at[pl.ds(base, CHUNK)], x_v)
            pltpu.sync_copy(seg_hbm.at[pl.ds(base, CHUNK)], seg_v)

        @pl.when(is_tail)
        def _():
            pltpu.sync_copy(x_hbm.at[pl.ds(base, TAIL)], x_v.at[pl.ds(0, TAIL)])
            pltpu.sync_copy(seg_hbm.at[pl.ds(base, TAIL)],
                            seg_v.at[pl.ds(0, TAIL)])
            dump16 = jnp.full((16,), DUMP_SEG, jnp.int32)
            for t in range(TAIL, CHUNK, 16):
                x_v[pl.ds(t, 16)] = zero16
                seg_v[pl.ds(t, 16)] = dump16

        pltpu.sync_copy(w_hbm, w_v)
        pltpu.sync_copy(b_hbm, b_v)

        def zbody(r, carry):
            for j in range(CGRP):
                acc_v[r, pl.ds(16 * j, 16)] = zero16
            return carry

        lax.fori_loop(0, SEGT, zbody, 0)

        w_regs = [w_v[pl.ds(16 * j, 16)] for j in range(CGRP)]
        b_regs = [b_v[pl.ds(16 * j, 16)] for j in range(CGRP)]
        csix = jnp.full((16,), 0.0625, jnp.float32)

        @plsc.parallel_loop(0, CHUNK // 16, 1, unroll=2)
        def gbody(g):
            x16 = x_v[pl.ds(g * 16, 16)]
            s16 = seg_v[pl.ds(g * 16, 16)]
            for lane in range(16):
                xi = x16[lane]
                si = s16[lane]
                for j in range(CGRP):
                    h = jnp.maximum(xi * w_regs[j] + b_regs[j], 0.0)
                    plsc.addupdate(acc_v.at[si, pl.ds(16 * j, 16)], h)
                plsc.addupdate(
                    acc_v.at[CNT_BASE + (si >> 3), pl.ds((si & 7) * 16, 16)],
                    csix)

        pltpu.sync_copy(acc_v, psum_hbm.at[wid])

    return k


def _combine(psum):
    def body(ps_ref, out_ref):
        s = jnp.sum(ps_ref[...], axis=0)
        sums = s[:NUM_GRAPHS]
        craw = s[CNT_BASE:CNT_BASE + NUM_GRAPHS // 8]
        rep = jnp.broadcast_to(
            craw[:, None, :], (NUM_GRAPHS // 8, 8, HIDDEN)
        ).reshape(NUM_GRAPHS, HIDDEN)
        lane_grp = lax.broadcasted_iota(jnp.int32, (NUM_GRAPHS, HIDDEN), 1) // 16
        seg_grp = lax.broadcasted_iota(jnp.int32, (NUM_GRAPHS, HIDDEN), 0) % 8
        cnt = jnp.sum(
            jnp.where(lane_grp == seg_grp, rep, 0.0), axis=1, keepdims=True)
        out_ref[...] = sums / jnp.maximum(cnt, 1.0)

    return pl.pallas_call(
        body,
        out_shape=jax.ShapeDtypeStruct((NUM_GRAPHS, HIDDEN), jnp.float32),
    )(psum)


def kernel(x, edge_index, batch, W, b):
    del edge_index
    x_flat = x.reshape(N_NODES)
    seg = batch.astype(jnp.int32)
    w_flat = W.reshape(HIDDEN)
    b_flat = b.reshape(HIDDEN)
    psum = _sc_partials()(x_flat, seg, w_flat, b_flat)
    return _combine(psum)


# speedup vs baseline: 1.1336x; 1.0101x over previous
import functools

import jax
import jax.numpy as jnp
from jax import lax
from jax.experimental import pallas as pl
from jax.experimental.pallas import tpu as pltpu
from jax.experimental.pallas import tpu_sc as plsc

N_NODES = 100000
HIDDEN = 128
NUM_GRAPHS = 512
NW = 32
CHUNK = 3136
TAIL = N_NODES - (NW - 1) * CHUNK
CGRP = HIDDEN // 16
CNT_BASE = NUM_GRAPHS
DUMP_SEG = 576
SEGT = 592


def _sc_partials():
    mesh = plsc.VectorSubcoreMesh(core_axis_name="c", subcore_axis_name="s")

    @functools.partial(
        pl.kernel,
        mesh=mesh,
        out_type=jax.ShapeDtypeStruct((NW, SEGT, HIDDEN), jnp.float32),
        scratch_types=[
            pltpu.VMEM((CHUNK,), jnp.float32),
            pltpu.VMEM((CHUNK,), jnp.int32),
            pltpu.VMEM((SEGT, HIDDEN), jnp.float32),
            pltpu.VMEM((HIDDEN,), jnp.float32),
            pltpu.VMEM((HIDDEN,), jnp.float32),
            pltpu.SemaphoreType.DMA,
            pltpu.SemaphoreType.DMA,
        ],
    )
    def k(x_hbm, seg_hbm, w_hbm, b_hbm, psum_hbm,
          x_v, seg_v, acc_v, w_v, b_v, semx, sems):
        wid = lax.axis_index("s") * 2 + lax.axis_index("c")
        base = wid * CHUNK
        is_tail = wid == NW - 1
        zero16 = jnp.zeros((16,), jnp.float32)

        def zero_acc():
            def zbody(r, carry):
                for j in range(CGRP):
                    acc_v[r, pl.ds(16 * j, 16)] = zero16
                return carry
            lax.fori_loop(0, SEGT, zbody, 0)

        @pl.when(jnp.logical_not(is_tail))
        def _():
            cx = pltpu.async_copy(x_hbm.at[pl.ds(base, CHUNK)], x_v, semx)
            cs = pltpu.async_copy(seg_hbm.at[pl.ds(base, CHUNK)], seg_v, sems)
            zero_acc()
            cx.wait()
            cs.wait()

        @pl.when(is_tail)
        def _():
            cx = pltpu.async_copy(
                x_hbm.at[pl.ds(base, TAIL)], x_v.at[pl.ds(0, TAIL)], semx)
            cs = pltpu.async_copy(
                seg_hbm.at[pl.ds(base, TAIL)], seg_v.at[pl.ds(0, TAIL)], sems)
            zero_acc()
            cx.wait()
            cs.wait()
            dump16 = jnp.full((16,), DUMP_SEG, jnp.int32)
            for t in range(TAIL, CHUNK, 16):
                x_v[pl.ds(t, 16)] = zero16
                seg_v[pl.ds(t, 16)] = dump16

        pltpu.sync_copy(w_hbm, w_v)
        pltpu.sync_copy(b_hbm, b_v)

        w_regs = [w_v[pl.ds(16 * j, 16)] for j in range(CGRP)]
        b_regs = [b_v[pl.ds(16 * j, 16)] for j in range(CGRP)]
        csix = jnp.full((16,), 0.0625, jnp.float32)

        @plsc.parallel_loop(0, CHUNK // 16, 1, unroll=4)
        def gbody(g):
            x16 = x_v[pl.ds(g * 16, 16)]
            s16 = seg_v[pl.ds(g * 16, 16)]
            for lane in range(16):
                xi = x16[lane]
                si = s16[lane]
                for j in range(CGRP):
                    h = jnp.maximum(xi * w_regs[j] + b_regs[j], 0.0)
                    plsc.addupdate(acc_v.at[si, pl.ds(16 * j, 16)], h)
                plsc.addupdate(
                    acc_v.at[CNT_BASE + (si >> 3), pl.ds((si & 7) * 16, 16)],
                    csix)

        pltpu.sync_copy(acc_v, psum_hbm.at[wid])

    return k


def _combine(psum):
    def body(ps_ref, out_ref):
        s = jnp.sum(ps_ref[...], axis=0)
        sums = s[:NUM_GRAPHS]
        craw = s[CNT_BASE:CNT_BASE + NUM_GRAPHS // 8]
        rep = jnp.broadcast_to(
            craw[:, None, :], (NUM_GRAPHS // 8, 8, HIDDEN)
        ).reshape(NUM_GRAPHS, HIDDEN)
        lane_grp = lax.broadcasted_iota(jnp.int32, (NUM_GRAPHS, HIDDEN), 1) // 16
        seg_grp = lax.broadcasted_iota(jnp.int32, (NUM_GRAPHS, HIDDEN), 0) % 8
        cnt = jnp.sum(
            jnp.where(lane_grp == seg_grp, rep, 0.0), axis=1, keepdims=True)
        out_ref[...] = sums / jnp.maximum(cnt, 1.0)

    return pl.pallas_call(
        body,
        out_shape=jax.ShapeDtypeStruct((NUM_GRAPHS, HIDDEN), jnp.float32),
    )(psum)


def kernel(x, edge_index, batch, W, b):
    del edge_index
    x_flat = x.reshape(N_NODES)
    seg = batch.astype(jnp.int32)
    w_flat = W.reshape(HIDDEN)
    b_flat = b.reshape(HIDDEN)
    psum = _sc_partials()(x_flat, seg, w_flat, b_flat)
    return _combine(psum)
